# Initial kernel scaffold; baseline (speedup 1.0000x reference)
import jax
import jax.numpy as jnp
from jax.experimental import pallas as pl


def kernel(x, edge_index, W1a, b1a, W1b, b1b, W2a, b2a, W2b, b2b, fcW, fcb):
    raise NotImplementedError("write your pallas kernel here")


# speedup vs baseline: 4.3116x; 4.3116x over previous
import functools

import jax
import jax.numpy as jnp
from jax import lax
from jax.experimental import pallas as pl
from jax.experimental.pallas import tpu as pltpu
from jax.experimental.pallas import tpu_sc as plsc

N_NODES = 10000
N_PAD = 10240
FSL = 128
N_EDGES = 160000
EDGE_K = 80


def _make_segsum(n_slabs):
  passes = n_slabs // 2
  e_tile = N_EDGES // 16
  chunks = e_tile // EDGE_K
  rows_t = N_PAD // 16

  mesh = plsc.VectorSubcoreMesh(core_axis_name="c", subcore_axis_name="s")

  @functools.partial(
      pl.kernel,
      out_type=jax.ShapeDtypeStruct((n_slabs, N_PAD, FSL), jnp.float32),
      mesh=mesh,
      scratch_types=[
          pltpu.VMEM((e_tile,), jnp.int32),
          pltpu.VMEM((chunks, EDGE_K), jnp.int32),
          pltpu.VMEM((EDGE_K, FSL), jnp.float32),
          pltpu.VMEM((16, FSL), jnp.float32),
          pltpu.VMEM_SHARED((N_PAD, FSL), jnp.float32),
      ],
  )
  def seg(table_hbm, src_hbm, dst_hbm, out_hbm, src_v, dst_v, stg_v, zero_v,
          acc_sh):
    c = lax.axis_index("c")
    s = lax.axis_index("s")
    pltpu.sync_copy(src_hbm.at[s], src_v)
    pltpu.sync_copy(dst_hbm.at[s], dst_v)
    zval = jnp.zeros((16,), jnp.float32)
    for r in range(16):
      for f in range(0, FSL, 16):
        zero_v[r, pl.ds(f, 16)] = zval

    for p in range(passes):
      slab = c * passes + p
      @pl.loop(0, rows_t, step=16)
      def _(r0):
        pltpu.sync_copy(zero_v, acc_sh.at[pl.ds(s * rows_t + r0, 16)])
      plsc.subcore_barrier()

      @pl.loop(0, chunks)
      def _(k):
        idx = src_v.at[pl.ds(k * EDGE_K, EDGE_K)]
        pltpu.sync_copy(table_hbm.at[slab].at[idx], stg_v)
        pltpu.sync_copy(stg_v, acc_sh.at[dst_v.at[k]], add=True)
      plsc.subcore_barrier()

      pltpu.sync_copy(
          acc_sh.at[pl.ds(s * rows_t, rows_t)],
          out_hbm.at[slab].at[pl.ds(s * rows_t, rows_t)])
      if p + 1 < passes:
        plsc.subcore_barrier()

  return seg


_segsum2 = _make_segsum(2)
_segsum4 = _make_segsum(4)


def _mlp1(x, agg_s, W1a, b1a, W1b, b1b):
  tn = 1000

  def body(x_ref, a_ref, wa, ba, wb, bb, h_ref):
    a = x_ref[...] + jnp.concatenate([a_ref[0], a_ref[1]], axis=1)
    t = jnp.dot(a, wa[...], preferred_element_type=jnp.float32) + ba[...]
    t = jnp.maximum(t, 0.0)
    h = jnp.dot(t, wb[...], preferred_element_type=jnp.float32) + bb[...]
    h_ref[...] = jnp.maximum(h, 0.0)

  return pl.pallas_call(
      body,
      grid=(N_NODES // tn,),
      in_specs=[
          pl.BlockSpec((tn, 256), lambda i: (i, 0)),
          pl.BlockSpec((2, tn, FSL), lambda i: (0, i, 0)),
          pl.BlockSpec((256, 512), lambda i: (0, 0)),
          pl.BlockSpec((1, 512), lambda i: (0, 0)),
          pl.BlockSpec((512, 512), lambda i: (0, 0)),
          pl.BlockSpec((1, 512), lambda i: (0, 0)),
      ],
      out_specs=pl.BlockSpec((tn, 512), lambda i: (i, 0)),
      out_shape=jax.ShapeDtypeStruct((N_NODES, 512), jnp.float32),
  )(x, agg_s, W1a, b1a.reshape(1, -1), W1b, b1b.reshape(1, -1))


def _mlp2_pool(h, agg_s, W2a, b2a, W2b, b2b, fcW, fcb):
  tn = 1000
  nsteps = N_NODES // tn

  def body(h_ref, a_ref, wa, ba, wb, bb, fw, fb, o_ref, acc):
    i = pl.program_id(0)
    a = h_ref[...] + jnp.concatenate(
        [a_ref[0], a_ref[1], a_ref[2], a_ref[3]], axis=1)
    t = jnp.dot(a, wa[...], preferred_element_type=jnp.float32) + ba[...]
    t = jnp.maximum(t, 0.0)
    h2 = jnp.dot(t, wb[...], preferred_element_type=jnp.float32) + bb[...]
    h2 = jnp.maximum(h2, 0.0)
    part = jnp.sum(h2, axis=0, keepdims=True)

    @pl.when(i == 0)
    def _():
      acc[...] = part

    @pl.when(i > 0)
    def _():
      acc[...] = acc[...] + part

    @pl.when(i == nsteps - 1)
    def _():
      o_ref[...] = jnp.dot(
          acc[...], fw[...], preferred_element_type=jnp.float32) + fb[...]

  return pl.pallas_call(
      body,
      grid=(nsteps,),
      in_specs=[
          pl.BlockSpec((tn, 512), lambda i: (i, 0)),
          pl.BlockSpec((4, tn, FSL), lambda i: (0, i, 0)),
          pl.BlockSpec((512, 512), lambda i: (0, 0)),
          pl.BlockSpec((1, 512), lambda i: (0, 0)),
          pl.BlockSpec((512, 512), lambda i: (0, 0)),
          pl.BlockSpec((1, 512), lambda i: (0, 0)),
          pl.BlockSpec((512, 256), lambda i: (0, 0)),
          pl.BlockSpec((1, 256), lambda i: (0, 0)),
      ],
      out_specs=pl.BlockSpec((1, 256), lambda i: (0, 0)),
      out_shape=jax.ShapeDtypeStruct((1, 256), jnp.float32),
      scratch_shapes=[pltpu.VMEM((1, 512), jnp.float32)],
  )(h, agg_s, W2a, b2a.reshape(1, -1), W2b, b2b.reshape(1, -1), fcW,
    fcb.reshape(1, -1))


def kernel(x, edge_index, W1a, b1a, W1b, b1b, W2a, b2a, W2b, b2b, fcW, fcb):
  src = edge_index[0].astype(jnp.int32).reshape(16, N_EDGES // 16)
  dst2 = edge_index[1].astype(jnp.int32).reshape(16, N_EDGES // 16 // EDGE_K,
                                                 EDGE_K)

  xs = x.reshape(N_NODES, 2, FSL).transpose(1, 0, 2)
  agg1 = _segsum2(xs, src, dst2)
  h = _mlp1(x, agg1, W1a, b1a, W1b, b1b)
  hs = h.reshape(N_NODES, 4, FSL).transpose(1, 0, 2)
  agg2 = _segsum4(hs, src, dst2)
  emb = _mlp2_pool(h, agg2, W2a, b2a, W2b, b2b, fcW, fcb)
  return emb.reshape(256)


# speedup vs baseline: 6.7325x; 1.5615x over previous
import functools

import jax
import jax.numpy as jnp
from jax import lax
from jax.experimental import pallas as pl
from jax.experimental.pallas import tpu as pltpu
from jax.experimental.pallas import tpu_sc as plsc

N_NODES = 10000
N_PAD = 10240
FSL = 128
N_EDGES = 160000
EDGE_K = 80


def _make_segsum(n_slabs, fsl):
  passes = n_slabs // 2
  e_tile = N_EDGES // 16
  chunks = e_tile // EDGE_K
  rows_t = N_PAD // 16
  nbuf = 2
  groups = chunks // nbuf

  mesh = plsc.VectorSubcoreMesh(core_axis_name="c", subcore_axis_name="s")

  @functools.partial(
      pl.kernel,
      out_type=jax.ShapeDtypeStruct((n_slabs, N_PAD, fsl), jnp.float32),
      mesh=mesh,
      scratch_types=[
          pltpu.VMEM((e_tile,), jnp.int32),
          pltpu.VMEM((chunks, EDGE_K), jnp.int32),
          pltpu.VMEM((nbuf, EDGE_K, fsl), jnp.float32),
          pltpu.VMEM_SHARED((N_PAD, fsl), jnp.float32),
      ] + [pltpu.SemaphoreType.DMA] * (2 * nbuf),
  )
  def seg(table_hbm, src_hbm, dst_hbm, out_hbm, src_v, dst_v, stg_v,
          acc_sh, *sems):
    gsem = sems[:nbuf]
    ssem = sems[nbuf:]
    c = lax.axis_index("c")
    s = lax.axis_index("s")
    pltpu.sync_copy(src_hbm.at[s], src_v)
    pltpu.sync_copy(dst_hbm.at[s], dst_v)
    def gather(slab, k, b, issue):
      d = pltpu.make_async_copy(
          table_hbm.at[slab].at[src_v.at[pl.ds(k * EDGE_K, EDGE_K)]],
          stg_v.at[b], gsem[b])
      d.start() if issue else d.wait()

    def scatter(k, b, issue):
      if issue:
        pltpu.async_copy(stg_v.at[b], acc_sh.at[dst_v.at[k]], ssem[b],
                         add=True)
      else:
        pltpu.make_async_copy(stg_v.at[b], acc_sh.at[dst_v.at[k]],
                              ssem[b]).wait()

    for p in range(passes):
      slab = c * passes + p
      zval = jnp.zeros((16,), jnp.float32)
      for r in range(16):
        for f in range(0, fsl, 16):
          stg_v[0, r, pl.ds(f, 16)] = zval

      @pl.loop(0, rows_t, step=16)
      def _(r0):
        pltpu.sync_copy(stg_v.at[0].at[pl.ds(0, 16)],
                        acc_sh.at[pl.ds(s * rows_t + r0, 16)])
      plsc.subcore_barrier()

      for b in range(nbuf):
        gather(slab, b, b, issue=True)

      @pl.loop(0, groups)
      def _(g):
        for b in range(nbuf):
          k = g * nbuf + b
          gather(slab, k, b, issue=False)
          scatter(k, b, issue=True)
          scatter(k, b, issue=False)
          @pl.when(k + nbuf < chunks)
          def _():
            gather(slab, k + nbuf, b, issue=True)
      for j in range(groups * nbuf, chunks):
        b = j % nbuf
        gather(slab, j, b, issue=False)
        scatter(j, b, issue=True)
        scatter(j, b, issue=False)
      plsc.subcore_barrier()

      pltpu.sync_copy(
          acc_sh.at[pl.ds(s * rows_t, rows_t)],
          out_hbm.at[slab].at[pl.ds(s * rows_t, rows_t)])
      if p + 1 < passes:
        plsc.subcore_barrier()

  return seg


_segsum2 = _make_segsum(2, 128)
_segsum4 = _make_segsum(4, 128)


def _mlp1(x, agg_s, W1a, b1a, W1b, b1b):
  tn = 1000

  def body(x_ref, a_ref, wa, ba, wb, bb, h_ref):
    a = x_ref[...] + jnp.concatenate([a_ref[0], a_ref[1]], axis=1)
    t = jnp.dot(a, wa[...], preferred_element_type=jnp.float32) + ba[...]
    t = jnp.maximum(t, 0.0)
    h = jnp.dot(t, wb[...], preferred_element_type=jnp.float32) + bb[...]
    h_ref[...] = jnp.maximum(h, 0.0)

  return pl.pallas_call(
      body,
      grid=(N_NODES // tn,),
      in_specs=[
          pl.BlockSpec((tn, 256), lambda i: (i, 0)),
          pl.BlockSpec((2, tn, FSL), lambda i: (0, i, 0)),
          pl.BlockSpec((256, 512), lambda i: (0, 0)),
          pl.BlockSpec((1, 512), lambda i: (0, 0)),
          pl.BlockSpec((512, 512), lambda i: (0, 0)),
          pl.BlockSpec((1, 512), lambda i: (0, 0)),
      ],
      out_specs=pl.BlockSpec((tn, 512), lambda i: (i, 0)),
      out_shape=jax.ShapeDtypeStruct((N_NODES, 512), jnp.float32),
  )(x, agg_s, W1a, b1a.reshape(1, -1), W1b, b1b.reshape(1, -1))


def _mlp2_pool(h, agg_s, W2a, b2a, W2b, b2b, fcW, fcb):
  tn = 1000
  nsteps = N_NODES // tn

  def body(h_ref, a_ref, wa, ba, wb, bb, fw, fb, o_ref, acc):
    i = pl.program_id(0)
    a = h_ref[...] + jnp.concatenate([a_ref[j] for j in range(4)], axis=1)
    t = jnp.dot(a, wa[...], preferred_element_type=jnp.float32) + ba[...]
    t = jnp.maximum(t, 0.0)
    h2 = jnp.dot(t, wb[...], preferred_element_type=jnp.float32) + bb[...]
    h2 = jnp.maximum(h2, 0.0)
    part = jnp.sum(h2, axis=0, keepdims=True)

    @pl.when(i == 0)
    def _():
      acc[...] = part

    @pl.when(i > 0)
    def _():
      acc[...] = acc[...] + part

    @pl.when(i == nsteps - 1)
    def _():
      o_ref[...] = jnp.dot(
          acc[...], fw[...], preferred_element_type=jnp.float32) + fb[...]

  return pl.pallas_call(
      body,
      grid=(nsteps,),
      in_specs=[
          pl.BlockSpec((tn, 512), lambda i: (i, 0)),
          pl.BlockSpec((4, tn, FSL), lambda i: (0, i, 0)),
          pl.BlockSpec((512, 512), lambda i: (0, 0)),
          pl.BlockSpec((1, 512), lambda i: (0, 0)),
          pl.BlockSpec((512, 512), lambda i: (0, 0)),
          pl.BlockSpec((1, 512), lambda i: (0, 0)),
          pl.BlockSpec((512, 256), lambda i: (0, 0)),
          pl.BlockSpec((1, 256), lambda i: (0, 0)),
      ],
      out_specs=pl.BlockSpec((1, 256), lambda i: (0, 0)),
      out_shape=jax.ShapeDtypeStruct((1, 256), jnp.float32),
      scratch_shapes=[pltpu.VMEM((1, 512), jnp.float32)],
  )(h, agg_s, W2a, b2a.reshape(1, -1), W2b, b2b.reshape(1, -1), fcW,
    fcb.reshape(1, -1))


def kernel(x, edge_index, W1a, b1a, W1b, b1b, W2a, b2a, W2b, b2b, fcW, fcb):
  src = edge_index[0].astype(jnp.int32).reshape(16, N_EDGES // 16)
  dst2 = edge_index[1].astype(jnp.int32).reshape(16, N_EDGES // 16 // EDGE_K,
                                                 EDGE_K)

  xs = x.reshape(N_NODES, 2, FSL).transpose(1, 0, 2)
  agg1 = _segsum2(xs, src, dst2)
  h = _mlp1(x, agg1, W1a, b1a, W1b, b1b)
  hs = h.reshape(N_NODES, 4, FSL).transpose(1, 0, 2)
  agg2 = _segsum4(hs, src, dst2)
  emb = _mlp2_pool(h, agg2, W2a, b2a, W2b, b2b, fcW, fcb)
  return emb.reshape(256)


# speedup vs baseline: 7.3324x; 1.0891x over previous
import functools

import jax
import jax.numpy as jnp
from jax import lax
from jax.experimental import pallas as pl
from jax.experimental.pallas import tpu as pltpu
from jax.experimental.pallas import tpu_sc as plsc

N_NODES = 10000
N_PAD = 10240
FSL = 128
N_EDGES = 160000
EDGE_K = 80


def _make_segsum(n_slabs, fsl):
  passes = n_slabs // 2
  e_tile = N_EDGES // 16
  chunks = e_tile // EDGE_K
  rows_t = N_PAD // 16
  nbuf = 2
  groups = chunks // nbuf

  mesh = plsc.VectorSubcoreMesh(core_axis_name="c", subcore_axis_name="s")

  @functools.partial(
      pl.kernel,
      out_type=jax.ShapeDtypeStruct((n_slabs, N_PAD, fsl), jnp.float32),
      mesh=mesh,
      scratch_types=[
          pltpu.VMEM((e_tile,), jnp.int32),
          pltpu.VMEM((chunks, EDGE_K), jnp.int32),
          pltpu.VMEM((nbuf, EDGE_K, fsl), jnp.float32),
          pltpu.VMEM_SHARED((N_PAD, fsl), jnp.float32),
      ] + [pltpu.SemaphoreType.DMA] * (2 * nbuf),
  )
  def seg(table_hbm, src_hbm, dst_hbm, out_hbm, src_v, dst_v, stg_v,
          acc_sh, *sems):
    gsem = sems[:nbuf]
    ssem = sems[nbuf:]
    c = lax.axis_index("c")
    s = lax.axis_index("s")
    pltpu.sync_copy(src_hbm.at[s], src_v)
    pltpu.sync_copy(dst_hbm.at[s], dst_v)
    def gather(slab, k, b, issue):
      d = pltpu.make_async_copy(
          table_hbm.at[slab].at[src_v.at[pl.ds(k * EDGE_K, EDGE_K)]],
          stg_v.at[b], gsem[b])
      d.start() if issue else d.wait()

    def scatter(k, b, issue):
      if issue:
        pltpu.async_copy(stg_v.at[b], acc_sh.at[dst_v.at[k]], ssem[b],
                         add=True)
      else:
        pltpu.make_async_copy(stg_v.at[b], acc_sh.at[dst_v.at[k]],
                              ssem[b]).wait()

    for p in range(passes):
      slab = c * passes + p
      zval = jnp.zeros((16,), jnp.float32)
      for r in range(16):
        for f in range(0, fsl, 16):
          stg_v[0, r, pl.ds(f, 16)] = zval

      @pl.loop(0, rows_t, step=16)
      def _(r0):
        pltpu.sync_copy(stg_v.at[0].at[pl.ds(0, 16)],
                        acc_sh.at[pl.ds(s * rows_t + r0, 16)])
      plsc.subcore_barrier()

      for b in range(nbuf):
        gather(slab, b, b, issue=True)

      @pl.loop(0, groups)
      def _(g):
        for b in range(nbuf):
          k = g * nbuf + b
          gather(slab, k, b, issue=False)
          scatter(k, b, issue=True)
          scatter(k, b, issue=False)
          @pl.when(k + nbuf < chunks)
          def _():
            gather(slab, k + nbuf, b, issue=True)
      for j in range(groups * nbuf, chunks):
        b = j % nbuf
        gather(slab, j, b, issue=False)
        scatter(j, b, issue=True)
        scatter(j, b, issue=False)
      plsc.subcore_barrier()

      pltpu.sync_copy(
          acc_sh.at[pl.ds(s * rows_t, rows_t)],
          out_hbm.at[slab].at[pl.ds(s * rows_t, rows_t)])
      if p + 1 < passes:
        plsc.subcore_barrier()

  return seg


_segsum2 = _make_segsum(2, 128)
_segsum4 = _make_segsum(4, 128)


def _mlp1(x, agg_s, W1a, b1a, W1b, b1b):
  tn = 1000

  def body(x_ref, a_ref, wa, ba, wb, bb, h_ref):
    a = x_ref[...] + jnp.concatenate([a_ref[0], a_ref[1]], axis=1)
    t = jnp.dot(a, wa[...], preferred_element_type=jnp.float32) + ba[...]
    t = jnp.maximum(t, 0.0)
    h = jnp.dot(t, wb[...], preferred_element_type=jnp.float32) + bb[...]
    h = jnp.maximum(h, 0.0)
    for j in range(4):
      h_ref[j] = h[:, j * FSL:(j + 1) * FSL]

  return pl.pallas_call(
      body,
      grid=(N_NODES // tn,),
      in_specs=[
          pl.BlockSpec((tn, 256), lambda i: (i, 0)),
          pl.BlockSpec((2, tn, FSL), lambda i: (0, i, 0)),
          pl.BlockSpec((256, 512), lambda i: (0, 0)),
          pl.BlockSpec((1, 512), lambda i: (0, 0)),
          pl.BlockSpec((512, 512), lambda i: (0, 0)),
          pl.BlockSpec((1, 512), lambda i: (0, 0)),
      ],
      out_specs=pl.BlockSpec((4, tn, FSL), lambda i: (0, i, 0)),
      out_shape=jax.ShapeDtypeStruct((4, N_NODES, FSL), jnp.float32),
  )(x, agg_s, W1a, b1a.reshape(1, -1), W1b, b1b.reshape(1, -1))


def _mlp2_pool(h, agg_s, W2a, b2a, W2b, b2b, fcW, fcb):
  tn = 1000
  nsteps = N_NODES // tn

  def body(h_ref, a_ref, wa, ba, wb, bb, fw, fb, o_ref, acc):
    i = pl.program_id(0)
    a = (jnp.concatenate([h_ref[j] for j in range(4)], axis=1)
         + jnp.concatenate([a_ref[j] for j in range(4)], axis=1))
    t = jnp.dot(a, wa[...], preferred_element_type=jnp.float32) + ba[...]
    t = jnp.maximum(t, 0.0)
    h2 = jnp.dot(t, wb[...], preferred_element_type=jnp.float32) + bb[...]
    h2 = jnp.maximum(h2, 0.0)
    part = jnp.sum(h2, axis=0, keepdims=True)

    @pl.when(i == 0)
    def _():
      acc[...] = part

    @pl.when(i > 0)
    def _():
      acc[...] = acc[...] + part

    @pl.when(i == nsteps - 1)
    def _():
      o_ref[...] = jnp.dot(
          acc[...], fw[...], preferred_element_type=jnp.float32) + fb[...]

  return pl.pallas_call(
      body,
      grid=(nsteps,),
      in_specs=[
          pl.BlockSpec((4, tn, FSL), lambda i: (0, i, 0)),
          pl.BlockSpec((4, tn, FSL), lambda i: (0, i, 0)),
          pl.BlockSpec((512, 512), lambda i: (0, 0)),
          pl.BlockSpec((1, 512), lambda i: (0, 0)),
          pl.BlockSpec((512, 512), lambda i: (0, 0)),
          pl.BlockSpec((1, 512), lambda i: (0, 0)),
          pl.BlockSpec((512, 256), lambda i: (0, 0)),
          pl.BlockSpec((1, 256), lambda i: (0, 0)),
      ],
      out_specs=pl.BlockSpec((1, 256), lambda i: (0, 0)),
      out_shape=jax.ShapeDtypeStruct((1, 256), jnp.float32),
      scratch_shapes=[pltpu.VMEM((1, 512), jnp.float32)],
  )(h, agg_s, W2a, b2a.reshape(1, -1), W2b, b2b.reshape(1, -1), fcW,
    fcb.reshape(1, -1))


def kernel(x, edge_index, W1a, b1a, W1b, b1b, W2a, b2a, W2b, b2b, fcW, fcb):
  src = edge_index[0].astype(jnp.int32).reshape(16, N_EDGES // 16)
  dst2 = edge_index[1].astype(jnp.int32).reshape(16, N_EDGES // 16 // EDGE_K,
                                                 EDGE_K)

  xs = x.reshape(N_NODES, 2, FSL).transpose(1, 0, 2)
  agg1 = _segsum2(xs, src, dst2)
  hs = _mlp1(x, agg1, W1a, b1a, W1b, b1b)
  agg2 = _segsum4(hs, src, dst2)
  emb = _mlp2_pool(hs, agg2, W2a, b2a, W2b, b2b, fcW, fcb)
  return emb.reshape(256)
